# Initial kernel scaffold; baseline (speedup 1.0000x reference)
import jax
import jax.numpy as jnp
from jax.experimental import pallas as pl


def kernel(x, edge_index, laplacian, weight, bias):
    raise NotImplementedError("write your pallas kernel here")


# speedup vs baseline: 3.9874x; 3.9874x over previous
import functools

import jax
import jax.numpy as jnp
from jax import lax
from jax.experimental import pallas as pl
from jax.experimental.pallas import tpu as pltpu
from jax.experimental.pallas import tpu_sc as plsc

N = 10000
E = 320000
D = 128
DF = 64
NC = 2
NS = 16
CH = 128
EPT = 20480
E_PAD = NS * EPT
NCH = EPT // CH
N_PAD = 10240
ROWS_PER_TILE = N_PAD // NS

R_TC = 1000


def _sc_prop(hflat, srcp, dstp, lapp):
    mesh = plsc.VectorSubcoreMesh(core_axis_name="c", subcore_axis_name="s")

    @functools.partial(
        pl.kernel,
        out_type=jax.ShapeDtypeStruct((NC * N_PAD, DF), jnp.float32),
        mesh=mesh,
        compiler_params=pltpu.CompilerParams(use_tc_tiling_on_sc=False),
        scratch_types=[
            pltpu.VMEM((NCH, CH), jnp.int32),
            pltpu.VMEM((NCH, CH), jnp.float32),
            pltpu.VMEM((CH,), jnp.int32),
            pltpu.VMEM((CH,), jnp.int32),
            pltpu.VMEM((CH, DF), jnp.float32),
            pltpu.VMEM((CH, DF), jnp.float32),
            pltpu.VMEM_SHARED((N_PAD, DF), jnp.float32),
            pltpu.SemaphoreType.DMA,
            pltpu.SemaphoreType.DMA,
            pltpu.SemaphoreType.DMA,
            pltpu.SemaphoreType.DMA,
        ],
    )
    def prop(h_hbm, src_hbm, dst_hbm, lap_hbm, part_hbm,
             srcl, lapl, dstb0, dstb1, rows0, rows1, acc,
             gsem0, gsem1, dsem0, dsem1):
        c = lax.axis_index("c")
        s = lax.axis_index("s")
        rbase = s * NCH

        pltpu.sync_copy(src_hbm.at[pl.ds(rbase, NCH)], srcl)
        pltpu.sync_copy(lap_hbm.at[pl.ds(rbase, NCH)], lapl)

        cbase = jnp.full((16,), c * N, jnp.int32)

        @pl.loop(0, NCH)
        def _shift(j):
            for i in range(CH // 16):
                sl = (j, pl.ds(i * 16, 16))
                srcl[sl] = srcl[sl] + cbase

        @pl.loop(0, CH)
        def _zero_rows(e):
            for j in range(DF // 16):
                rows0[e, pl.ds(j * 16, 16)] = jnp.zeros((16,), jnp.float32)

        for q in range(ROWS_PER_TILE // CH):
            pltpu.sync_copy(
                rows0,
                acc.at[pl.ds(s * ROWS_PER_TILE + q * CH, CH)],
            )
        plsc.subcore_barrier()

        def fire(t, dstb, rows, gsem, dsem):
            pltpu.async_copy(dst_hbm.at[rbase + t], dstb, dsem)
            pltpu.async_copy(h_hbm.at[srcl.at[t]], rows, gsem)

        fire(0, dstb0, rows0, gsem0, dsem0)
        fire(1, dstb1, rows1, gsem1, dsem1)

        @pl.loop(0, NCH, step=2)
        def _chunks(t0):
            for b, (dstb, rows, gsem, dsem) in enumerate(
                ((dstb0, rows0, gsem0, dsem0), (dstb1, rows1, gsem1, dsem1))
            ):
                t = t0 + b
                pltpu.make_async_copy(h_hbm.at[pl.ds(0, CH)], rows, gsem).wait()

                @pl.loop(0, CH, step=16)
                def _scale(e0):
                    lvec = lapl[t, pl.ds(e0, 16)]
                    for i in range(16):
                        wv = jnp.full((16,), lvec[i], jnp.float32)
                        for j in range(DF // 16):
                            sl = pl.ds(j * 16, 16)
                            rows[e0 + i, sl] = rows[e0 + i, sl] * wv

                pltpu.make_async_copy(dst_hbm.at[rbase], dstb, dsem).wait()
                pltpu.sync_copy(rows, acc.at[dstb], add=True)

                @pl.when(t + 2 < NCH)
                def _next():
                    fire(t + 2, dstb, rows, gsem, dsem)

        plsc.subcore_barrier()
        pltpu.sync_copy(
            acc.at[pl.ds(s * ROWS_PER_TILE, ROWS_PER_TILE)],
            part_hbm.at[pl.ds(c * N_PAD + s * ROWS_PER_TILE, ROWS_PER_TILE)],
        )

    return prop(hflat, srcp, dstp, lapp)


def _tc_init(x, w0, bias2):
    def body(x_ref, w_ref, b_ref, o_ref):
        o_ref[...] = jnp.dot(
            x_ref[...], w_ref[...],
            preferred_element_type=jnp.float32,
            precision=lax.Precision.HIGHEST,
        ) + b_ref[...]

    return pl.pallas_call(
        body,
        grid=(N // R_TC,),
        in_specs=[
            pl.BlockSpec((R_TC, D), lambda i: (i, 0)),
            pl.BlockSpec((D, D), lambda i: (0, 0)),
            pl.BlockSpec((1, D), lambda i: (0, 0)),
        ],
        out_specs=pl.BlockSpec((R_TC, D), lambda i: (i, 0)),
        out_shape=jax.ShapeDtypeStruct((N, D), jnp.float32),
    )(x, w0, bias2)


def _tc_step(parts, tx_prev, out_in, wk2, a, b, do_relu):
    def body(p_ref, tp_ref, oin_ref, w_ref, tx_ref, o_ref):
        t = a * p_ref[...]
        if b != 0.0:
            t = t + b * tp_ref[...]
        tx_ref[...] = t
        o = oin_ref[...] + jnp.dot(
            t[0], w_ref[0],
            preferred_element_type=jnp.float32,
            precision=lax.Precision.HIGHEST,
        ) + jnp.dot(
            t[1], w_ref[1],
            preferred_element_type=jnp.float32,
            precision=lax.Precision.HIGHEST,
        )
        if do_relu:
            o = jnp.maximum(o, 0.0)
        o_ref[...] = o

    return pl.pallas_call(
        body,
        grid=(N // R_TC,),
        in_specs=[
            pl.BlockSpec((NC, R_TC, DF), lambda i: (0, i, 0)),
            pl.BlockSpec((NC, R_TC, DF), lambda i: (0, i, 0)),
            pl.BlockSpec((R_TC, D), lambda i: (i, 0)),
            pl.BlockSpec((NC, DF, D), lambda i: (0, 0, 0)),
        ],
        out_specs=[
            pl.BlockSpec((NC, R_TC, DF), lambda i: (0, i, 0)),
            pl.BlockSpec((R_TC, D), lambda i: (i, 0)),
        ],
        out_shape=[
            jax.ShapeDtypeStruct((NC, N, DF), jnp.float32),
            jax.ShapeDtypeStruct((N, D), jnp.float32),
        ],
    )(parts, tx_prev, out_in, wk2)


def kernel(x, edge_index, laplacian, weight, bias):
    src = edge_index[0]
    dst = edge_index[1]
    pad = E_PAD - E
    srcp = jnp.concatenate([src, jnp.zeros((pad,), src.dtype)]).reshape(-1, CH)
    dstp = jnp.concatenate([dst, jnp.zeros((pad,), dst.dtype)]).reshape(-1, CH)
    lapp = jnp.concatenate(
        [laplacian, jnp.zeros((pad,), laplacian.dtype)]
    ).reshape(-1, CH)
    bias2 = bias.reshape(1, D)

    out = _tc_init(x, weight[0], bias2)
    xs = x.reshape(N, NC, DF).transpose(1, 0, 2)
    tx_m2, tx_m1 = xs, xs
    for k in range(1, weight.shape[0]):
        parts = _sc_prop(
            tx_m1.reshape(NC * N, DF), srcp, dstp, lapp
        ).reshape(NC, N_PAD, DF)
        a, b = (1.0, 0.0) if k == 1 else (2.0, -1.0)
        wk2 = weight[k].reshape(NC, DF, D)
        tx_new, out = _tc_step(
            parts, tx_m2, out, wk2, a, b, do_relu=(k == weight.shape[0] - 1)
        )
        tx_m2, tx_m1 = tx_m1, tx_new
    return out
